# Initial kernel scaffold; baseline (speedup 1.0000x reference)
import jax
import jax.numpy as jnp
from jax.experimental import pallas as pl


def kernel(raw_feature, Wq, bq, Wk, bk, Wv, bv, Wo, bo, ln_g, ln_b):
    raise NotImplementedError("write your pallas kernel here")


# speedup vs baseline: 2.6846x; 2.6846x over previous
import jax
import jax.numpy as jnp
from jax.experimental import pallas as pl
from jax.experimental.pallas import tpu as pltpu

D = 1024
H = 8
HD = D // H
N = 512
B = 16


def _fused_layer_kernel(x_ref, wqkv_ref, bqkv_ref, wo_ref, bo_ref, ln_ref,
                        out_ref):
    x = x_ref[...]
    xb = x.astype(jnp.bfloat16)
    qkv = jax.lax.dot_general(
        xb, wqkv_ref[...],
        (((1,), (0,)), ((), ())),
        preferred_element_type=jnp.float32) + bqkv_ref[...]

    scale = 1.0 / (HD ** 0.5)
    ctx_parts = []
    for h in range(H):
        q = qkv[:, h * HD:(h + 1) * HD]
        k = qkv[:, D + h * HD:D + (h + 1) * HD]
        v = qkv[:, 2 * D + h * HD:2 * D + (h + 1) * HD]
        s = jax.lax.dot_general(
            q.astype(jnp.bfloat16), k.astype(jnp.bfloat16),
            (((1,), (1,)), ((), ())),
            preferred_element_type=jnp.float32) * scale
        m = jnp.max(s, axis=1, keepdims=True)
        e = jnp.exp(s - m)
        p = e * (1.0 / jnp.sum(e, axis=1, keepdims=True))
        ctx_parts.append(jax.lax.dot_general(
            p.astype(jnp.bfloat16), v.astype(jnp.bfloat16),
            (((1,), (0,)), ((), ())),
            preferred_element_type=jnp.float32))
    ctx = jnp.concatenate(ctx_parts, axis=1)

    h_out = jax.lax.dot_general(
        ctx.astype(jnp.bfloat16), wo_ref[...],
        (((1,), (0,)), ((), ())),
        preferred_element_type=jnp.float32) + bo_ref[...]
    y = h_out + x
    mu = jnp.mean(y, axis=1, keepdims=True)
    yc = y - mu
    var = jnp.mean(yc * yc, axis=1, keepdims=True)
    y = yc * jax.lax.rsqrt(var + 1e-12) * ln_ref[0:1, :] + ln_ref[1:2, :]
    norm = jnp.sqrt(jnp.sum(y * y, axis=1, keepdims=True)) + 1e-12
    out_ref[...] = y * (1.0 / norm)


def kernel(raw_feature, Wq, bq, Wk, bk, Wv, bv, Wo, bo, ln_g, ln_b):
    x2d = raw_feature.reshape(B * N, D)
    wqkv = jnp.concatenate([Wq, Wk, Wv], axis=1).astype(jnp.bfloat16)
    bqkv = jnp.concatenate([bq, bk, bv]).reshape(1, 3 * D)
    ln = jnp.stack([ln_g, ln_b], axis=0)

    out = pl.pallas_call(
        _fused_layer_kernel,
        grid=(B,),
        in_specs=[
            pl.BlockSpec((N, D), lambda b: (b, 0)),
            pl.BlockSpec((D, 3 * D), lambda b: (0, 0)),
            pl.BlockSpec((1, 3 * D), lambda b: (0, 0)),
            pl.BlockSpec((D, D), lambda b: (0, 0)),
            pl.BlockSpec((1, D), lambda b: (0, 0)),
            pl.BlockSpec((2, D), lambda b: (0, 0)),
        ],
        out_specs=pl.BlockSpec((N, D), lambda b: (b, 0)),
        out_shape=jax.ShapeDtypeStruct((B * N, D), jnp.float32),
        compiler_params=pltpu.CompilerParams(
            dimension_semantics=("arbitrary",),
        ),
    )(x2d, wqkv, bqkv, Wo.astype(jnp.bfloat16), bo.reshape(1, D), ln)
    return out.reshape(B, N, D)


# speedup vs baseline: 2.8273x; 1.0531x over previous
import math

import jax
import jax.numpy as jnp
from jax.experimental import pallas as pl
from jax.experimental.pallas import tpu as pltpu

D = 1024
H = 8
HD = D // H
N = 512
B = 16


def _fused_layer_kernel(x_ref, wqkv_ref, bqkv_ref, wo_ref, bo_ref, ln_ref,
                        out_ref, ctx_ref):
    x = x_ref[...]
    qkv = (jax.lax.dot_general(
        x.astype(jnp.bfloat16), wqkv_ref[...],
        (((1,), (0,)), ((), ())),
        preferred_element_type=jnp.float32)
        + bqkv_ref[...]).astype(jnp.bfloat16)

    for h in range(H):
        q = qkv[:, h * HD:(h + 1) * HD]
        k = qkv[:, D + h * HD:D + (h + 1) * HD]
        v = qkv[:, 2 * D + h * HD:2 * D + (h + 1) * HD]
        s = jax.lax.dot_general(
            q, k, (((1,), (1,)), ((), ())),
            preferred_element_type=jnp.float32)
        m = jnp.max(s, axis=1, keepdims=True)
        e = jnp.exp2(s - m)
        r = 1.0 / jnp.sum(e, axis=1, keepdims=True)
        c = jax.lax.dot_general(
            e.astype(jnp.bfloat16), v, (((1,), (0,)), ((), ())),
            preferred_element_type=jnp.float32)
        ctx_ref[:, h * HD:(h + 1) * HD] = (c * r).astype(jnp.bfloat16)

    h_out = jax.lax.dot_general(
        ctx_ref[...], wo_ref[...],
        (((1,), (0,)), ((), ())),
        preferred_element_type=jnp.float32) + bo_ref[...]
    y = h_out + x
    mu = jnp.mean(y, axis=1, keepdims=True)
    yc = y - mu
    var = jnp.mean(yc * yc, axis=1, keepdims=True)
    y = yc * jax.lax.rsqrt(var + 1e-12) * ln_ref[0:1, :] + ln_ref[1:2, :]
    norm = jnp.sqrt(jnp.sum(y * y, axis=1, keepdims=True)) + 1e-12
    out_ref[...] = y * (1.0 / norm)


def kernel(raw_feature, Wq, bq, Wk, bk, Wv, bv, Wo, bo, ln_g, ln_b):
    x2d = raw_feature.reshape(B * N, D)
    qscale = math.log2(math.e) / math.sqrt(HD)
    wqkv = jnp.concatenate(
        [Wq * qscale, Wk, Wv], axis=1).astype(jnp.bfloat16)
    bqkv = jnp.concatenate(
        [bq * qscale, bk, bv]).reshape(1, 3 * D).astype(jnp.bfloat16)
    ln = jnp.stack([ln_g, ln_b], axis=0)

    out = pl.pallas_call(
        _fused_layer_kernel,
        grid=(B,),
        in_specs=[
            pl.BlockSpec((N, D), lambda b: (b, 0)),
            pl.BlockSpec((D, 3 * D), lambda b: (0, 0)),
            pl.BlockSpec((1, 3 * D), lambda b: (0, 0)),
            pl.BlockSpec((D, D), lambda b: (0, 0)),
            pl.BlockSpec((1, D), lambda b: (0, 0)),
            pl.BlockSpec((2, D), lambda b: (0, 0)),
        ],
        out_specs=pl.BlockSpec((N, D), lambda b: (b, 0)),
        out_shape=jax.ShapeDtypeStruct((B * N, D), jnp.float32),
        scratch_shapes=[pltpu.VMEM((N, D), jnp.bfloat16)],
        compiler_params=pltpu.CompilerParams(
            dimension_semantics=("arbitrary",),
        ),
    )(x2d, wqkv, bqkv, Wo.astype(jnp.bfloat16), bo.reshape(1, D), ln)
    return out.reshape(B, N, D)
